# baseline (speedup 1.0000x reference)
import functools

import jax
import jax.numpy as jnp
from jax import lax
from jax.experimental import pallas as pl
from jax.experimental.pallas import tpu as pltpu, tpu_sc as plsc

B = 16384
K = 32
NC = 2
NS = 16
NW = NC * NS
BPW = B // NW
CHUNK = 128
NCHUNK = BPW // CHUNK


def _body(xu_hbm, xi_hbm, w_hbm, h_hbm, out_hbm,
          idx_u, idx_i, u_rows, h_rows, out_v, sem):
  wid = lax.axis_index("s") * NC + lax.axis_index("c")
  row0 = wid * NCHUNK

  pltpu.sync_copy(xu_hbm.at[pl.ds(row0, NCHUNK)], idx_u)
  pltpu.sync_copy(xi_hbm.at[pl.ds(row0, NCHUNK)], idx_i)

  copies = []
  for j in range(NCHUNK):
    dst_u = u_rows.at[pl.ds(j * CHUNK, CHUNK)]
    dst_i = h_rows.at[pl.ds(j * CHUNK, CHUNK)]
    copies.append(pltpu.async_copy(w_hbm.at[idx_u.at[j]], dst_u, sem))
    copies.append(pltpu.async_copy(h_hbm.at[idx_i.at[j]], dst_i, sem))
  for c in copies:
    c.wait()

  iota = lax.iota(jnp.int32, 16)

  def group(g, _):
    rows = g * 16 + iota
    acc = jnp.zeros((16,), jnp.float32)
    for k in range(K):
      cols = jnp.full((16,), k, jnp.int32)
      u = plsc.load_gather(u_rows, [rows, cols])
      v = plsc.load_gather(h_rows, [rows, cols])
      acc = acc + u * v
    plsc.store_scatter(out_v, [rows], acc)
    return _

  lax.fori_loop(0, BPW // 16, group, None)

  pltpu.sync_copy(out_v, out_hbm.at[pl.ds(wid * BPW, BPW)])


def kernel(x_user, x_item, W, H):
  xu = x_user.astype(jnp.int32).reshape(B // CHUNK, CHUNK)
  xi = x_item.astype(jnp.int32).reshape(B // CHUNK, CHUNK)

  mesh = plsc.VectorSubcoreMesh(core_axis_name="c", subcore_axis_name="s")
  k = functools.partial(
      pl.kernel,
      out_type=jax.ShapeDtypeStruct((B,), jnp.float32),
      mesh=mesh,
      compiler_params=pltpu.CompilerParams(
          needs_layout_passes=False, use_tc_tiling_on_sc=False),
      scratch_types=[
          pltpu.VMEM((NCHUNK, CHUNK), jnp.int32),
          pltpu.VMEM((NCHUNK, CHUNK), jnp.int32),
          pltpu.VMEM((BPW, K), jnp.float32),
          pltpu.VMEM((BPW, K), jnp.float32),
          pltpu.VMEM((BPW,), jnp.float32),
          pltpu.SemaphoreType.DMA,
      ],
  )(_body)
  return k(xu, xi, W, H)
